# baseline (speedup 1.0000x reference)
import functools

import jax
import jax.numpy as jnp
from jax import lax
from jax.experimental import pallas as pl
from jax.experimental.pallas import tpu as pltpu
from jax.experimental.pallas import tpu_sc as plsc

NUM_OBJ = 1000
NUM_OBJ_P = 1024
NUM_OBJ_CLS = 151
NUM_REL = 20000
NUM_REL_P = 20480
NUM_REL_CLS = 51
ROW_W = 128


def _rowsum8(e):
    n, w = e.shape
    padw = ((w + 7) // 8) * 8
    if padw != w:
        e = jnp.concatenate([e, jnp.zeros((n, padw - w), jnp.float32)], axis=1)
    acc = e[:, 0:8]
    for i in range(1, padw // 8):
        acc = acc + e[:, i * 8:(i + 1) * 8]
    t = acc[:, 0:4] + acc[:, 4:8]
    t = t[:, 0:2] + t[:, 2:4]
    return t[:, 0:1] + t[:, 1:2]


def _obj_body(obj_ref, os_ref, op_ref):
    obj = obj_ref[...]
    m = jnp.max(obj, axis=1, keepdims=True)
    e = jnp.exp(obj - m)
    s = _rowsum8(e)
    prob = e / s
    cols = lax.broadcasted_iota(jnp.int32, prob.shape, 1)
    pm = jnp.where(cols >= 1, prob, -1.0)
    osc = jnp.max(pm, axis=1)
    opred = jnp.min(jnp.where(pm == osc[:, None], cols, 10**9), axis=1)
    os_ref[...] = osc.reshape(8, 128)
    op_ref[...] = opred.reshape(8, 128)


REL_BLK = 2000


def _rel_body(rel_ref, pair_ref, comb_ref, rs_ref):
    rel = rel_ref[...]
    rm = jnp.max(rel, axis=1, keepdims=True)
    re_ = jnp.exp(rel - rm)
    rs = _rowsum8(re_)
    rprob = re_ / rs
    rcols = lax.broadcasted_iota(jnp.int32, rprob.shape, 1)
    rpm = jnp.where(rcols >= 1, rprob, -1.0)
    rsc = jnp.max(rpm, axis=1)
    rcls = jnp.min(jnp.where(rpm == rsc[:, None], rcols, 10**9), axis=1)

    pair_f = lax.bitcast_convert_type(pair_ref[...], jnp.float32)
    rcls_f = lax.bitcast_convert_type(rcls, jnp.float32)[:, None]
    comb_ref[:, :NUM_REL_CLS] = rprob
    comb_ref[:, NUM_REL_CLS:NUM_REL_CLS + 2] = pair_f
    comb_ref[:, NUM_REL_CLS + 2:NUM_REL_CLS + 3] = rcls_f
    comb_ref[:, NUM_REL_CLS + 3:] = jnp.zeros(
        (REL_BLK, ROW_W - NUM_REL_CLS - 3), jnp.float32)
    rs_ref[...] = rsc[:, None]


def _prep(rel_logit, obj_logit_p, rel_pair_idx):
    osc, opred = pl.pallas_call(
        _obj_body,
        out_shape=[
            jax.ShapeDtypeStruct((8, 128), jnp.float32),
            jax.ShapeDtypeStruct((8, 128), jnp.int32),
        ],
    )(obj_logit_p)
    nblk = NUM_REL // REL_BLK
    comb, rsc = pl.pallas_call(
        _rel_body,
        grid=(nblk,),
        in_specs=[
            pl.BlockSpec((REL_BLK, NUM_REL_CLS), lambda i: (i, 0)),
            pl.BlockSpec((REL_BLK, 2), lambda i: (i, 0)),
        ],
        out_specs=[
            pl.BlockSpec((REL_BLK, ROW_W), lambda i: (i, 0)),
            pl.BlockSpec((REL_BLK, 1), lambda i: (i, 0)),
        ],
        out_shape=[
            jax.ShapeDtypeStruct((NUM_REL, ROW_W), jnp.float32),
            jax.ShapeDtypeStruct((NUM_REL, 1), jnp.float32),
        ],
    )(rel_logit, rel_pair_idx)
    return comb, rsc, osc, opred


_NW = 32
_BPW = NUM_REL_P // _NW
_CHUNK = 128
_NCH = _BPW // _CHUNK


def _permute_rows(table, idx_1d):
    mesh = plsc.VectorSubcoreMesh(core_axis_name="c", subcore_axis_name="s")

    @functools.partial(
        pl.kernel, mesh=mesh,
        out_type=jax.ShapeDtypeStruct((NUM_REL_P, ROW_W), jnp.float32),
        scratch_types=[
            pltpu.VMEM((_BPW,), jnp.int32),
            pltpu.VMEM((_BPW, ROW_W), jnp.float32),
            pltpu.SemaphoreType.DMA,
        ],
    )
    def k(table_hbm, idx_hbm, out_hbm, idx_v, rows_v, sem):
        wid = lax.axis_index("s") * 2 + lax.axis_index("c")
        pltpu.sync_copy(idx_hbm.at[pl.ds(wid * _BPW, _BPW)], idx_v)
        copies = [
            pltpu.async_copy(table_hbm.at[idx_v.at[pl.ds(j * _CHUNK, _CHUNK)]],
                             rows_v.at[pl.ds(j * _CHUNK, _CHUNK)], sem)
            for j in range(_NCH)
        ]
        for c in copies:
            c.wait()
        pltpu.sync_copy(rows_v, out_hbm.at[pl.ds(wid * _BPW, _BPW)])

    return k(table, idx_1d)


def kernel(rel_logit, obj_logit, rel_pair_idx):
    obj_logit_p = jnp.pad(obj_logit, ((0, NUM_OBJ_P - NUM_OBJ), (0, 0)))
    comb, rs, osc, opred = _prep(rel_logit, obj_logit_p, rel_pair_idx)
    obj_scores = osc.reshape(-1)[:NUM_OBJ]
    obj_pred = opred.reshape(-1)[:NUM_OBJ]

    triple = (rs.reshape(-1) * obj_scores[rel_pair_idx[:, 0]]
              * obj_scores[rel_pair_idx[:, 1]])
    sorting_idx = jnp.argsort(-triple)
    idx_p = jnp.concatenate(
        [sorting_idx.astype(jnp.int32),
         jnp.zeros((NUM_REL_P - NUM_REL,), jnp.int32)])
    sortedbuf = _permute_rows(comb, idx_p)

    rel_pair_idx_sorted = lax.bitcast_convert_type(sortedbuf[:NUM_REL, 51:53], jnp.int32)
    rel_class_prob_sorted = sortedbuf[:NUM_REL, :51]
    rel_labels = lax.bitcast_convert_type(sortedbuf[:NUM_REL, 53], jnp.int32)
    return (obj_pred, obj_scores, rel_pair_idx_sorted, rel_class_prob_sorted, rel_labels)


# speedup vs baseline: 1.4259x; 1.4259x over previous
import functools

import jax
import jax.numpy as jnp
from jax import lax
from jax.experimental import pallas as pl
from jax.experimental.pallas import tpu as pltpu
from jax.experimental.pallas import tpu_sc as plsc

NUM_OBJ = 1000
NUM_OBJ_P = 1024
NUM_OBJ_CLS = 151
NUM_REL = 20000
NUM_REL_P = 20480
NUM_REL_CLS = 51
ROW_W = 128


def _rowsum8(e):
    n, w = e.shape
    padw = ((w + 7) // 8) * 8
    if padw != w:
        e = jnp.concatenate([e, jnp.zeros((n, padw - w), jnp.float32)], axis=1)
    acc = e[:, 0:8]
    for i in range(1, padw // 8):
        acc = acc + e[:, i * 8:(i + 1) * 8]
    t = acc[:, 0:4] + acc[:, 4:8]
    t = t[:, 0:2] + t[:, 2:4]
    return t[:, 0:1] + t[:, 1:2]


def _obj_body(obj_ref, os_ref, op_ref):
    obj = obj_ref[...]
    m = jnp.max(obj, axis=1, keepdims=True)
    e = jnp.exp(obj - m)
    s = _rowsum8(e)
    prob = e / s
    cols = lax.broadcasted_iota(jnp.int32, prob.shape, 1)
    pm = jnp.where(cols >= 1, prob, -1.0)
    osc = jnp.max(pm, axis=1)
    opred = jnp.min(jnp.where(pm == osc[:, None], cols, 10**9), axis=1)
    os_ref[...] = osc.reshape(8, 128)
    op_ref[...] = opred.reshape(8, 128)


REL_BLK = 2000


def _rel_body(rel_ref, pair_ref, comb_ref, rs_ref):
    rel = rel_ref[...]
    rm = jnp.max(rel, axis=1, keepdims=True)
    re_ = jnp.exp(rel - rm)
    rs = _rowsum8(re_)
    rprob = re_ / rs
    rcols = lax.broadcasted_iota(jnp.int32, rprob.shape, 1)
    rpm = jnp.where(rcols >= 1, rprob, -1.0)
    rsc = jnp.max(rpm, axis=1)
    rcls = jnp.min(jnp.where(rpm == rsc[:, None], rcols, 10**9), axis=1)

    pair_f = lax.bitcast_convert_type(pair_ref[...], jnp.float32)
    rcls_f = lax.bitcast_convert_type(rcls, jnp.float32)[:, None]
    comb_ref[:, :NUM_REL_CLS] = rprob
    comb_ref[:, NUM_REL_CLS:NUM_REL_CLS + 2] = pair_f
    comb_ref[:, NUM_REL_CLS + 2:NUM_REL_CLS + 3] = rcls_f
    comb_ref[:, NUM_REL_CLS + 3:] = jnp.zeros(
        (REL_BLK, ROW_W - NUM_REL_CLS - 3), jnp.float32)
    rs_ref[...] = rsc[:, None]


def _prep(rel_logit, obj_logit_p, rel_pair_idx):
    osc, opred = pl.pallas_call(
        _obj_body,
        out_shape=[
            jax.ShapeDtypeStruct((8, 128), jnp.float32),
            jax.ShapeDtypeStruct((8, 128), jnp.int32),
        ],
    )(obj_logit_p)
    nblk = NUM_REL // REL_BLK
    comb, rsc = pl.pallas_call(
        _rel_body,
        grid=(nblk,),
        in_specs=[
            pl.BlockSpec((REL_BLK, NUM_REL_CLS), lambda i: (i, 0)),
            pl.BlockSpec((REL_BLK, 2), lambda i: (i, 0)),
        ],
        out_specs=[
            pl.BlockSpec((REL_BLK, ROW_W), lambda i: (i, 0)),
            pl.BlockSpec((REL_BLK, 1), lambda i: (i, 0)),
        ],
        out_shape=[
            jax.ShapeDtypeStruct((NUM_REL, ROW_W), jnp.float32),
            jax.ShapeDtypeStruct((NUM_REL, 1), jnp.float32),
        ],
    )(rel_logit, rel_pair_idx)
    return comb, rsc, osc, opred


_SORT_T = 16
_SORT_EPT = NUM_REL_P // _SORT_T
_SORT_V = _SORT_EPT // 16
_NB = 256


def _sc_argsort(rs_pad, p0_pad, p1_pad, osc_flat):
    mesh = plsc.VectorSubcoreMesh(core_axis_name="c", subcore_axis_name="s")

    @functools.partial(
        pl.kernel, mesh=mesh,
        out_type=jax.ShapeDtypeStruct((NUM_REL_P,), jnp.int32),
        compiler_params=pltpu.CompilerParams(needs_layout_passes=False),
        scratch_types=[
            pltpu.VMEM((_SORT_EPT,), jnp.float32),
            pltpu.VMEM((_SORT_EPT,), jnp.int32),
            pltpu.VMEM((_SORT_EPT,), jnp.int32),
            pltpu.VMEM((NUM_OBJ_P,), jnp.float32),
            pltpu.VMEM((_SORT_EPT,), jnp.int32),
            pltpu.VMEM((_SORT_EPT,), jnp.int32),
            pltpu.VMEM((16 * _NB,), jnp.int32),
            pltpu.VMEM((_NB,), jnp.int32),
            pltpu.VMEM((16 * _NB,), jnp.int32),
            pltpu.VMEM((_NB,), jnp.int32),
            pltpu.VMEM((_SORT_EPT // 128, 128), jnp.int32),
            pltpu.VMEM_SHARED((NUM_REL_P,), jnp.int32),
            pltpu.VMEM_SHARED((NUM_REL_P,), jnp.int32),
            pltpu.VMEM_SHARED((NUM_REL_P,), jnp.int32),
            pltpu.VMEM_SHARED((NUM_REL_P,), jnp.int32),
            pltpu.VMEM_SHARED((16 * _NB,), jnp.int32),
        ],
    )
    def k(rs_hbm, p0_hbm, p1_hbm, osc_hbm, out_hbm,
          rs_v, p0_v, p1_v, osc_v, key_v, idx_v, hist_v, th_v, ght_v,
          cur_v, dst_v, sk0, si0, sk1, si1, gh):
        core = lax.axis_index("c")
        t = lax.axis_index("s")

        @pl.when(core == 0)
        def _body():
            base = t * _SORT_EPT
            lane = lax.iota(jnp.int32, 16)
            ones = jnp.ones((16,), jnp.int32)

            pltpu.sync_copy(rs_hbm.at[pl.ds(base, _SORT_EPT)], rs_v)
            pltpu.sync_copy(p0_hbm.at[pl.ds(base, _SORT_EPT)], p0_v)
            pltpu.sync_copy(p1_hbm.at[pl.ds(base, _SORT_EPT)], p1_v)
            pltpu.sync_copy(osc_hbm, osc_v)

            def keybuild(v, c):
                sl = pl.ds(v * 16, 16)
                s0 = plsc.load_gather(osc_v, [p0_v[sl]])
                s1 = plsc.load_gather(osc_v, [p1_v[sl]])
                tri = (rs_v[sl] * s0) * s1
                bitsv = plsc.bitcast(tri, jnp.int32)
                gidx = base + v * 16 + lane
                key_v[sl] = jnp.where(gidx >= NUM_REL, jnp.int32(-1),
                                      jnp.int32(0x7FFFFFFE) - bitsv)
                idx_v[sl] = gidx
                return c
            lax.fori_loop(0, _SORT_V, keybuild, 0)

            bufs = [(sk0, si0), (sk1, si1)]
            for p in range(4):
                shiftv = jnp.full((16,), 8 * p, jnp.int32)
                if p > 0:
                    rk, ri = bufs[(p + 1) % 2]
                    pltpu.sync_copy(rk.at[pl.ds(base, _SORT_EPT)], key_v)
                    pltpu.sync_copy(ri.at[pl.ds(base, _SORT_EPT)], idx_v)
                wk, wi = bufs[p % 2]

                def zeroh(j, c):
                    hist_v[pl.ds(j * 16, 16)] = jnp.zeros((16,), jnp.int32)
                    return c
                lax.fori_loop(0, 16 * _NB // 16, zeroh, 0)

                def histb(v, c):
                    kv = key_v[pl.ds(v * 16, 16)]
                    d = jnp.bitwise_and(lax.shift_right_logical(kv, shiftv), 255)
                    plsc.addupdate_scatter(hist_v, [lane * _NB + d], ones)
                    return c
                lax.fori_loop(0, _SORT_V, histb, 0)

                def lred(c, carry):
                    def inner(l, acc):
                        return acc + hist_v[pl.ds(l * _NB + c * 16, 16)]
                    th_v[pl.ds(c * 16, 16)] = lax.fori_loop(
                        0, 16, inner, jnp.zeros((16,), jnp.int32))
                    return carry
                lax.fori_loop(0, _NB // 16, lred, 0)

                pltpu.sync_copy(th_v, gh.at[pl.ds(t * _NB, _NB)])
                plsc.subcore_barrier()
                pltpu.sync_copy(gh, ght_v)

                def totpre(c, carry):
                    def inner(l, tp):
                        tot, pre = tp
                        h = ght_v[pl.ds(l * _NB + c * 16, 16)]
                        return (tot + h, pre + h * jnp.where(l < t, 1, 0))
                    tot, pre = lax.fori_loop(
                        0, _SORT_T, inner,
                        (jnp.zeros((16,), jnp.int32), jnp.zeros((16,), jnp.int32)))
                    th_v[pl.ds(c * 16, 16)] = tot
                    cur_v[pl.ds(c * 16, 16)] = pre
                    return carry
                lax.fori_loop(0, _NB // 16, totpre, 0)

                def scan(c, s):
                    seg = th_v[pl.ds(c * 16, 16)]
                    inc = plsc.cumsum(seg)
                    cur_v[pl.ds(c * 16, 16)] = (
                        cur_v[pl.ds(c * 16, 16)] + (inc - seg) + s)
                    return s + jnp.sum(seg)
                lax.fori_loop(0, _NB // 16, scan, jnp.int32(0))

                for j in range(_SORT_EPT // 128):
                    def pbody(u, c, j=j):
                        v = j * 8 + u
                        kv = key_v[pl.ds(v * 16, 16)]
                        d = jnp.bitwise_and(
                            lax.shift_right_logical(kv, shiftv), 255)
                        w = jnp.zeros((16,), jnp.int32)
                        aft = jnp.zeros((16,), jnp.int32)
                        for u16 in range(16):
                            du = jnp.sum(jnp.where(lane == u16, d, 0))
                            m = d == du
                            w = w + jnp.where(m & (lane > u16), 1, 0)
                            aft = aft + jnp.where(m & (lane < u16), 1, 0)
                        dest = plsc.load_gather(cur_v, [d]) + w
                        plsc.addupdate_scatter(cur_v, [d], w + 1,
                                               mask=(aft == 0))
                        dst_v[j, pl.ds(u * 16, 16)] = dest
                        return c
                    lax.fori_loop(0, 8, pbody, 0)

                for j in range(_SORT_EPT // 128):
                    if p < 3:
                        pltpu.sync_copy(key_v.at[pl.ds(j * 128, 128)],
                                        wk.at[dst_v.at[j]])
                    pltpu.sync_copy(idx_v.at[pl.ds(j * 128, 128)],
                                    wi.at[dst_v.at[j]])
                plsc.subcore_barrier()

            pltpu.sync_copy(si1.at[pl.ds(base, _SORT_EPT)], idx_v)
            pltpu.sync_copy(idx_v, out_hbm.at[pl.ds(base, _SORT_EPT)])

    return k(rs_pad, p0_pad, p1_pad, osc_flat)


_NW = 32
_BPW = NUM_REL_P // _NW
_CHUNK = 128
_NCH = _BPW // _CHUNK


def _permute_rows(table, idx_1d):
    mesh = plsc.VectorSubcoreMesh(core_axis_name="c", subcore_axis_name="s")

    @functools.partial(
        pl.kernel, mesh=mesh,
        out_type=jax.ShapeDtypeStruct((NUM_REL_P, ROW_W), jnp.float32),
        scratch_types=[
            pltpu.VMEM((_BPW,), jnp.int32),
            pltpu.VMEM((_BPW, ROW_W), jnp.float32),
            pltpu.SemaphoreType.DMA,
        ],
    )
    def k(table_hbm, idx_hbm, out_hbm, idx_v, rows_v, sem):
        wid = lax.axis_index("s") * 2 + lax.axis_index("c")
        pltpu.sync_copy(idx_hbm.at[pl.ds(wid * _BPW, _BPW)], idx_v)
        copies = [
            pltpu.async_copy(table_hbm.at[idx_v.at[pl.ds(j * _CHUNK, _CHUNK)]],
                             rows_v.at[pl.ds(j * _CHUNK, _CHUNK)], sem)
            for j in range(_NCH)
        ]
        for c in copies:
            c.wait()
        pltpu.sync_copy(rows_v, out_hbm.at[pl.ds(wid * _BPW, _BPW)])

    return k(table, idx_1d)


def kernel(rel_logit, obj_logit, rel_pair_idx):
    obj_logit_p = jnp.pad(obj_logit, ((0, NUM_OBJ_P - NUM_OBJ), (0, 0)))
    comb, rs, osc, opred = _prep(rel_logit, obj_logit_p, rel_pair_idx)
    obj_scores = osc.reshape(-1)[:NUM_OBJ]
    obj_pred = opred.reshape(-1)[:NUM_OBJ]

    pad = NUM_REL_P - NUM_REL
    rs_pad = jnp.concatenate([rs.reshape(-1), jnp.full((pad,), -1.0, jnp.float32)])
    p0_pad = jnp.concatenate([rel_pair_idx[:, 0], jnp.zeros((pad,), jnp.int32)])
    p1_pad = jnp.concatenate([rel_pair_idx[:, 1], jnp.zeros((pad,), jnp.int32)])
    sorting_idx = _sc_argsort(rs_pad, p0_pad, p1_pad, osc.reshape(-1))
    idx_p = jnp.minimum(sorting_idx, NUM_REL - 1)
    sortedbuf = _permute_rows(comb, idx_p)

    rel_pair_idx_sorted = lax.bitcast_convert_type(sortedbuf[:NUM_REL, 51:53], jnp.int32)
    rel_class_prob_sorted = sortedbuf[:NUM_REL, :51]
    rel_labels = lax.bitcast_convert_type(sortedbuf[:NUM_REL, 53], jnp.int32)
    return (obj_pred, obj_scores, rel_pair_idx_sorted, rel_class_prob_sorted, rel_labels)


# speedup vs baseline: 1.6060x; 1.1263x over previous
import functools

import jax
import jax.numpy as jnp
from jax import lax
from jax.experimental import pallas as pl
from jax.experimental.pallas import tpu as pltpu
from jax.experimental.pallas import tpu_sc as plsc

NUM_OBJ = 1000
NUM_OBJ_P = 1024
NUM_OBJ_CLS = 151
NUM_REL = 20000
NUM_REL_P = 20480
NUM_REL_CLS = 51
ROW_W = 128


def _rowsum8(e):
    n, w = e.shape
    padw = ((w + 7) // 8) * 8
    if padw != w:
        e = jnp.concatenate([e, jnp.zeros((n, padw - w), jnp.float32)], axis=1)
    acc = e[:, 0:8]
    for i in range(1, padw // 8):
        acc = acc + e[:, i * 8:(i + 1) * 8]
    t = acc[:, 0:4] + acc[:, 4:8]
    t = t[:, 0:2] + t[:, 2:4]
    return t[:, 0:1] + t[:, 1:2]


def _obj_body(obj_ref, os_ref, op_ref):
    obj = obj_ref[...]
    m = jnp.max(obj, axis=1, keepdims=True)
    e = jnp.exp(obj - m)
    s = _rowsum8(e)
    prob = e / s
    cols = lax.broadcasted_iota(jnp.int32, prob.shape, 1)
    pm = jnp.where(cols >= 1, prob, -1.0)
    osc = jnp.max(pm, axis=1)
    opred = jnp.min(jnp.where(pm == osc[:, None], cols, 10**9), axis=1)
    os_ref[...] = osc.reshape(8, 128)
    op_ref[...] = opred.reshape(8, 128)


REL_BLK = 2000


def _rel_body(rel_ref, comb_ref, rs_ref, rc_ref):
    rel = rel_ref[...]
    rm = jnp.max(rel, axis=1, keepdims=True)
    re_ = jnp.exp(rel - rm)
    rs = _rowsum8(re_)
    rprob = re_ / rs
    rcols = lax.broadcasted_iota(jnp.int32, rprob.shape, 1)
    rpm = jnp.where(rcols >= 1, rprob, -1.0)
    rsc = jnp.max(rpm, axis=1)
    rcls = jnp.min(jnp.where(rpm == rsc[:, None], rcols, 10**9), axis=1)

    comb_ref[:, :NUM_REL_CLS] = rprob
    rs_ref[...] = rsc[:, None]
    rc_ref[...] = rcls[:, None]


def _prep(rel_logit, obj_logit_p, rel_pair_idx):
    osc, opred = pl.pallas_call(
        _obj_body,
        out_shape=[
            jax.ShapeDtypeStruct((8, 128), jnp.float32),
            jax.ShapeDtypeStruct((8, 128), jnp.int32),
        ],
    )(obj_logit_p)
    nblk = NUM_REL // REL_BLK
    comb, rsc, rcls = pl.pallas_call(
        _rel_body,
        grid=(nblk,),
        in_specs=[
            pl.BlockSpec((REL_BLK, NUM_REL_CLS), lambda i: (i, 0)),
        ],
        out_specs=[
            pl.BlockSpec((REL_BLK, ROW_W), lambda i: (i, 0)),
            pl.BlockSpec((REL_BLK, 1), lambda i: (i, 0)),
            pl.BlockSpec((REL_BLK, 1), lambda i: (i, 0)),
        ],
        out_shape=[
            jax.ShapeDtypeStruct((NUM_REL, ROW_W), jnp.float32),
            jax.ShapeDtypeStruct((NUM_REL, 1), jnp.float32),
            jax.ShapeDtypeStruct((NUM_REL, 1), jnp.int32),
        ],
    )(rel_logit)
    return comb, rsc, rcls, osc, opred


_SORT_T = 16
_SORT_EPT = NUM_REL_P // _SORT_T
_SORT_V = _SORT_EPT // 16
_NB = 256


def _sc_argsort(rs_pad, p0_pad, p1_pad, lbl_pad, osc_flat):
    mesh = plsc.VectorSubcoreMesh(core_axis_name="c", subcore_axis_name="s")

    @functools.partial(
        pl.kernel, mesh=mesh,
        out_type=[
            jax.ShapeDtypeStruct((NUM_REL_P,), jnp.int32),
            jax.ShapeDtypeStruct((NUM_REL * 4,), jnp.int32),
        ],
        compiler_params=pltpu.CompilerParams(needs_layout_passes=False),
        scratch_types=[
            pltpu.VMEM((_SORT_EPT,), jnp.float32),
            pltpu.VMEM((NUM_REL_P,), jnp.int32),
            pltpu.VMEM((NUM_REL_P,), jnp.int32),
            pltpu.VMEM((NUM_REL_P,), jnp.int32),
            pltpu.VMEM((NUM_OBJ_P,), jnp.float32),
            pltpu.VMEM((_SORT_EPT,), jnp.int32),
            pltpu.VMEM((_SORT_EPT,), jnp.int32),
            pltpu.VMEM((4 * _SORT_EPT,), jnp.int32),
            pltpu.VMEM((16 * _NB,), jnp.int32),
            pltpu.VMEM((_NB,), jnp.int32),
            pltpu.VMEM((16 * _NB,), jnp.int32),
            pltpu.VMEM((_NB,), jnp.int32),
            pltpu.VMEM((_SORT_EPT // 128, 128), jnp.int32),
            pltpu.VMEM_SHARED((NUM_REL_P,), jnp.int32),
            pltpu.VMEM_SHARED((NUM_REL_P,), jnp.int32),
            pltpu.VMEM_SHARED((NUM_REL_P,), jnp.int32),
            pltpu.VMEM_SHARED((NUM_REL_P,), jnp.int32),
            pltpu.VMEM_SHARED((16 * _NB,), jnp.int32),
        ],
    )
    def k(rs_hbm, p0_hbm, p1_hbm, lbl_hbm, osc_hbm, out_hbm, misc_hbm,
          rs_v, p0f, p1f, lblf, osc_v, key_v, idx_v, mf_v, hist_v, th_v,
          ght_v, cur_v, dst_v, sk0, si0, sk1, si1, gh):
        core = lax.axis_index("c")
        t = lax.axis_index("s")

        @pl.when(core == 0)
        def _body():
            base = t * _SORT_EPT
            lane = lax.iota(jnp.int32, 16)
            ones = jnp.ones((16,), jnp.int32)

            pltpu.sync_copy(rs_hbm.at[pl.ds(base, _SORT_EPT)], rs_v)
            pltpu.sync_copy(p0_hbm, p0f)
            pltpu.sync_copy(p1_hbm, p1f)
            pltpu.sync_copy(lbl_hbm, lblf)
            pltpu.sync_copy(osc_hbm, osc_v)

            def keybuild(v, c):
                sl = pl.ds(v * 16, 16)
                slg = pl.ds(base + v * 16, 16)
                s0 = plsc.load_gather(osc_v, [p0f[slg]])
                s1 = plsc.load_gather(osc_v, [p1f[slg]])
                tri = (rs_v[sl] * s0) * s1
                bitsv = plsc.bitcast(tri, jnp.int32)
                gidx = base + v * 16 + lane
                key_v[sl] = jnp.where(gidx >= NUM_REL, jnp.int32(-1),
                                      jnp.int32(0x7FFFFFFE) - bitsv)
                idx_v[sl] = gidx
                return c
            lax.fori_loop(0, _SORT_V, keybuild, 0)

            bufs = [(sk0, si0), (sk1, si1)]
            for p in range(4):
                shiftv = jnp.full((16,), 8 * p, jnp.int32)
                if p > 0:
                    rk, ri = bufs[(p + 1) % 2]
                    pltpu.sync_copy(rk.at[pl.ds(base, _SORT_EPT)], key_v)
                    pltpu.sync_copy(ri.at[pl.ds(base, _SORT_EPT)], idx_v)
                wk, wi = bufs[p % 2]

                def zeroh(j, c):
                    hist_v[pl.ds(j * 16, 16)] = jnp.zeros((16,), jnp.int32)
                    return c
                lax.fori_loop(0, 16 * _NB // 16, zeroh, 0)

                def histb(v, c):
                    kv = key_v[pl.ds(v * 16, 16)]
                    d = jnp.bitwise_and(lax.shift_right_logical(kv, shiftv), 255)
                    plsc.addupdate_scatter(hist_v, [lane * _NB + d], ones)
                    return c
                lax.fori_loop(0, _SORT_V, histb, 0)

                def lred(c, carry):
                    def inner(l, acc):
                        return acc + hist_v[pl.ds(l * _NB + c * 16, 16)]
                    th_v[pl.ds(c * 16, 16)] = lax.fori_loop(
                        0, 16, inner, jnp.zeros((16,), jnp.int32))
                    return carry
                lax.fori_loop(0, _NB // 16, lred, 0)

                pltpu.sync_copy(th_v, gh.at[pl.ds(t * _NB, _NB)])
                plsc.subcore_barrier()
                pltpu.sync_copy(gh, ght_v)

                def totpre(c, carry):
                    def inner(l, tp):
                        tot, pre = tp
                        h = ght_v[pl.ds(l * _NB + c * 16, 16)]
                        return (tot + h, pre + h * jnp.where(l < t, 1, 0))
                    tot, pre = lax.fori_loop(
                        0, _SORT_T, inner,
                        (jnp.zeros((16,), jnp.int32), jnp.zeros((16,), jnp.int32)))
                    th_v[pl.ds(c * 16, 16)] = tot
                    cur_v[pl.ds(c * 16, 16)] = pre
                    return carry
                lax.fori_loop(0, _NB // 16, totpre, 0)

                def scan(c, s):
                    seg = th_v[pl.ds(c * 16, 16)]
                    inc = plsc.cumsum(seg)
                    cur_v[pl.ds(c * 16, 16)] = (
                        cur_v[pl.ds(c * 16, 16)] + (inc - seg) + s)
                    return s + jnp.sum(seg)
                lax.fori_loop(0, _NB // 16, scan, jnp.int32(0))

                for j in range(_SORT_EPT // 128):
                    def pbody(u, c, j=j):
                        v = j * 8 + u
                        kv = key_v[pl.ds(v * 16, 16)]
                        d = jnp.bitwise_and(
                            lax.shift_right_logical(kv, shiftv), 255)
                        w = jnp.zeros((16,), jnp.int32)
                        tot = jnp.zeros((16,), jnp.int32)
                        for u16 in range(16):
                            du = jnp.sum(jnp.where(lane == u16, d, 0))
                            m = jnp.where(d == du, 1, 0)
                            w = w + jnp.where(lane > u16, m, 0)
                            tot = tot + m
                        dest = plsc.load_gather(cur_v, [d]) + w
                        plsc.addupdate_scatter(cur_v, [d], w + 1,
                                               mask=(w + 1 == tot))
                        dst_v[j, pl.ds(u * 16, 16)] = dest
                        return c
                    lax.fori_loop(0, 8, pbody, 0)

                for j in range(_SORT_EPT // 128):
                    if p < 3:
                        pltpu.sync_copy(key_v.at[pl.ds(j * 128, 128)],
                                        wk.at[dst_v.at[j]])
                    pltpu.sync_copy(idx_v.at[pl.ds(j * 128, 128)],
                                    wi.at[dst_v.at[j]])
                plsc.subcore_barrier()

            pltpu.sync_copy(si1.at[pl.ds(base, _SORT_EPT)], idx_v)

            def clampi(v, c):
                sl = pl.ds(v * 16, 16)
                idx_v[sl] = jnp.minimum(idx_v[sl], jnp.int32(NUM_REL - 1))
                return c
            lax.fori_loop(0, _SORT_V, clampi, 0)
            pltpu.sync_copy(idx_v, out_hbm.at[pl.ds(base, _SORT_EPT)])

            lane4 = lax.shift_right_logical(lane, 2)
            lanem = jnp.bitwise_and(lane, 3)

            def miscb(st, c):
                se = plsc.load_gather(idx_v, [st * 4 + lane4])
                a = plsc.load_gather(p0f, [se])
                b = plsc.load_gather(p1f, [se])
                l2 = plsc.load_gather(lblf, [se])
                mf_v[pl.ds(st * 16, 16)] = jnp.where(
                    lanem == 0, a,
                    jnp.where(lanem == 1, b,
                              jnp.where(lanem == 2, l2, 0)))
                return c
            lax.fori_loop(0, _SORT_EPT * 4 // 16, miscb, 0)

            live4 = (NUM_REL - (_SORT_T - 1) * _SORT_EPT) * 4

            @pl.when(t < _SORT_T - 1)
            def _mw():
                pltpu.sync_copy(mf_v, misc_hbm.at[pl.ds(base * 4, _SORT_EPT * 4)])

            @pl.when(t == _SORT_T - 1)
            def _mwt():
                pltpu.sync_copy(mf_v.at[pl.ds(0, live4)],
                                misc_hbm.at[pl.ds(base * 4, live4)])

    return k(rs_pad, p0_pad, p1_pad, lbl_pad, osc_flat)


_NW = 32
_BPW = NUM_REL_P // _NW
_CHUNK = 128
_NCH = _BPW // _CHUNK


def _permute_rows(table, idx_1d):
    mesh = plsc.VectorSubcoreMesh(core_axis_name="c", subcore_axis_name="s")
    tail = NUM_REL - (_NW - 1) * _BPW

    @functools.partial(
        pl.kernel, mesh=mesh,
        out_type=jax.ShapeDtypeStruct((NUM_REL, ROW_W), jnp.float32),
        scratch_types=[
            pltpu.VMEM((_BPW,), jnp.int32),
            pltpu.VMEM((_BPW, ROW_W), jnp.float32),
            pltpu.SemaphoreType.DMA,
        ],
    )
    def k(table_hbm, idx_hbm, prob_hbm, idx_v, rows_v, sem):
        wid = lax.axis_index("s") * 2 + lax.axis_index("c")
        base = wid * _BPW
        pltpu.sync_copy(idx_hbm.at[pl.ds(base, _BPW)], idx_v)
        copies = [
            pltpu.async_copy(table_hbm.at[idx_v.at[pl.ds(j * _CHUNK, _CHUNK)]],
                             rows_v.at[pl.ds(j * _CHUNK, _CHUNK)], sem)
            for j in range(_NCH)
        ]
        for c in copies:
            c.wait()

        @pl.when(wid < _NW - 1)
        def _full():
            pltpu.sync_copy(rows_v, prob_hbm.at[pl.ds(base, _BPW)])

        @pl.when(wid == _NW - 1)
        def _tail():
            pltpu.sync_copy(rows_v.at[pl.ds(0, tail)],
                            prob_hbm.at[pl.ds(base, tail)])

    return k(table, idx_1d)


def kernel(rel_logit, obj_logit, rel_pair_idx):
    obj_logit_p = jnp.pad(obj_logit, ((0, NUM_OBJ_P - NUM_OBJ), (0, 0)))
    comb, rs, rcls, osc, opred = _prep(rel_logit, obj_logit_p, rel_pair_idx)
    obj_scores = osc.reshape(-1)[:NUM_OBJ]
    obj_pred = opred.reshape(-1)[:NUM_OBJ]

    pad = NUM_REL_P - NUM_REL
    rs_pad = jnp.concatenate([rs.reshape(-1), jnp.full((pad,), -1.0, jnp.float32)])
    p0_pad = jnp.concatenate([rel_pair_idx[:, 0], jnp.zeros((pad,), jnp.int32)])
    p1_pad = jnp.concatenate([rel_pair_idx[:, 1], jnp.zeros((pad,), jnp.int32)])
    lbl_pad = jnp.concatenate([rcls.reshape(-1), jnp.zeros((pad,), jnp.int32)])
    sorting_idx, misc = _sc_argsort(rs_pad, p0_pad, p1_pad, lbl_pad,
                                    osc.reshape(-1))
    probs_full = _permute_rows(comb, sorting_idx)

    rel_class_prob_sorted = probs_full[:, :NUM_REL_CLS]
    misc2 = misc.reshape(NUM_REL, 4)
    rel_pair_idx_sorted = misc2[:, 0:2]
    rel_labels = misc2[:, 2]
    return (obj_pred, obj_scores, rel_pair_idx_sorted, rel_class_prob_sorted, rel_labels)


# speedup vs baseline: 1.7468x; 1.0876x over previous
import functools

import jax
import jax.numpy as jnp
from jax import lax
from jax.experimental import pallas as pl
from jax.experimental.pallas import tpu as pltpu
from jax.experimental.pallas import tpu_sc as plsc

NUM_OBJ = 1000
NUM_OBJ_P = 1024
NUM_OBJ_CLS = 151
NUM_REL = 20000
NUM_REL_P = 20480
NUM_REL_CLS = 51
ROW_W = 128


def _rowsum8(e):
    n, w = e.shape
    padw = ((w + 7) // 8) * 8
    if padw != w:
        e = jnp.concatenate([e, jnp.zeros((n, padw - w), jnp.float32)], axis=1)
    acc = e[:, 0:8]
    for i in range(1, padw // 8):
        acc = acc + e[:, i * 8:(i + 1) * 8]
    t = acc[:, 0:4] + acc[:, 4:8]
    t = t[:, 0:2] + t[:, 2:4]
    return t[:, 0:1] + t[:, 1:2]


def _obj_body(obj_ref, os_ref, op_ref):
    obj = obj_ref[...]
    m = jnp.max(obj, axis=1, keepdims=True)
    e = jnp.exp(obj - m)
    s = _rowsum8(e)
    prob = e / s
    cols = lax.broadcasted_iota(jnp.int32, prob.shape, 1)
    pm = jnp.where(cols >= 1, prob, -1.0)
    osc = jnp.max(pm, axis=1)
    opred = jnp.min(jnp.where(pm == osc[:, None], cols, 10**9), axis=1)
    os_ref[...] = osc.reshape(8, 128)
    op_ref[...] = opred.reshape(8, 128)


REL_BLK = 2048


def _rel_body(rel_ref, comb_ref, rs_ref, rc_ref):
    rel = rel_ref[...]
    rm = jnp.max(rel, axis=1, keepdims=True)
    re_ = jnp.exp(rel - rm)
    rs = _rowsum8(re_)
    rprob = re_ / rs
    rcols = lax.broadcasted_iota(jnp.int32, rprob.shape, 1)
    rpm = jnp.where(rcols >= 1, rprob, -1.0)
    rsc = jnp.max(rpm, axis=1)
    rcls = jnp.min(jnp.where(rpm == rsc[:, None], rcols, 10**9), axis=1)

    comb_ref[:, :NUM_REL_CLS] = rprob
    rs_ref[...] = rsc
    rc_ref[...] = rcls


def _prep(rel_logit, obj_logit_p, rel_pair_idx):
    osc, opred = pl.pallas_call(
        _obj_body,
        out_shape=[
            jax.ShapeDtypeStruct((8, 128), jnp.float32),
            jax.ShapeDtypeStruct((8, 128), jnp.int32),
        ],
    )(obj_logit_p)
    nblk = (NUM_REL + REL_BLK - 1) // REL_BLK
    comb, rsc, rcls = pl.pallas_call(
        _rel_body,
        grid=(nblk,),
        in_specs=[
            pl.BlockSpec((REL_BLK, NUM_REL_CLS), lambda i: (i, 0)),
        ],
        out_specs=[
            pl.BlockSpec((REL_BLK, ROW_W), lambda i: (i, 0)),
            pl.BlockSpec((REL_BLK,), lambda i: (i,)),
            pl.BlockSpec((REL_BLK,), lambda i: (i,)),
        ],
        out_shape=[
            jax.ShapeDtypeStruct((NUM_REL, ROW_W), jnp.float32),
            jax.ShapeDtypeStruct((NUM_REL,), jnp.float32),
            jax.ShapeDtypeStruct((NUM_REL,), jnp.int32),
        ],
    )(rel_logit)
    return comb, rsc, rcls, osc, opred


_SORT_T = 16
_SORT_EPT = NUM_REL_P // _SORT_T
_SORT_V = _SORT_EPT // 16
_NB = 256


def _sc_argsort(rs_pad, p0_pad, p1_pad, lbl_pad, osc_flat):
    mesh = plsc.VectorSubcoreMesh(core_axis_name="c", subcore_axis_name="s")

    @functools.partial(
        pl.kernel, mesh=mesh,
        out_type=[
            jax.ShapeDtypeStruct((NUM_REL_P,), jnp.int32),
            jax.ShapeDtypeStruct((NUM_REL,), jnp.int32),
            jax.ShapeDtypeStruct((NUM_REL,), jnp.int32),
            jax.ShapeDtypeStruct((NUM_REL,), jnp.int32),
        ],
        compiler_params=pltpu.CompilerParams(needs_layout_passes=False),
        scratch_types=[
            pltpu.VMEM((_SORT_EPT,), jnp.float32),
            pltpu.VMEM((NUM_REL_P,), jnp.int32),
            pltpu.VMEM((NUM_REL_P,), jnp.int32),
            pltpu.VMEM((NUM_REL_P,), jnp.int32),
            pltpu.VMEM((NUM_OBJ_P,), jnp.float32),
            pltpu.VMEM((_SORT_EPT,), jnp.int32),
            pltpu.VMEM((_SORT_EPT,), jnp.int32),
            pltpu.VMEM((_SORT_EPT,), jnp.int32),
            pltpu.VMEM((_SORT_EPT,), jnp.int32),
            pltpu.VMEM((_SORT_EPT,), jnp.int32),
            pltpu.VMEM((16 * _NB,), jnp.int32),
            pltpu.VMEM((_NB,), jnp.int32),
            pltpu.VMEM((16 * _NB,), jnp.int32),
            pltpu.VMEM((_NB,), jnp.int32),
            pltpu.VMEM((_SORT_EPT // 128, 128), jnp.int32),
            pltpu.VMEM_SHARED((NUM_REL_P,), jnp.int32),
            pltpu.VMEM_SHARED((NUM_REL_P,), jnp.int32),
            pltpu.VMEM_SHARED((NUM_REL_P,), jnp.int32),
            pltpu.VMEM_SHARED((NUM_REL_P,), jnp.int32),
            pltpu.VMEM_SHARED((16 * _NB,), jnp.int32),
        ],
    )
    def k(rs_hbm, p0_hbm, p1_hbm, lbl_hbm, osc_hbm,
          out_hbm, p0s_hbm, p1s_hbm, lbls_hbm,
          rs_v, p0f, p1f, lblf, osc_v, key_v, idx_v, p0s_v, p1s_v, lbls_v,
          hist_v, th_v, ght_v, cur_v, dst_v, sk0, si0, sk1, si1, gh):
        core = lax.axis_index("c")
        t = lax.axis_index("s")

        @pl.when(core == 0)
        def _body():
            base = t * _SORT_EPT
            lane = lax.iota(jnp.int32, 16)
            ones = jnp.ones((16,), jnp.int32)

            pltpu.sync_copy(rs_hbm.at[pl.ds(base, _SORT_EPT)], rs_v)
            pltpu.sync_copy(p0_hbm, p0f)
            pltpu.sync_copy(p1_hbm, p1f)
            pltpu.sync_copy(lbl_hbm, lblf)
            pltpu.sync_copy(osc_hbm, osc_v)

            def keybuild(v, c):
                sl = pl.ds(v * 16, 16)
                slg = pl.ds(base + v * 16, 16)
                s0 = plsc.load_gather(osc_v, [p0f[slg]])
                s1 = plsc.load_gather(osc_v, [p1f[slg]])
                tri = (rs_v[sl] * s0) * s1
                bitsv = plsc.bitcast(tri, jnp.int32)
                gidx = base + v * 16 + lane
                key_v[sl] = jnp.where(gidx >= NUM_REL, jnp.int32(-1),
                                      jnp.int32(0x7FFFFFFE) - bitsv)
                idx_v[sl] = gidx
                return c
            lax.fori_loop(0, _SORT_V, keybuild, 0)

            bufs = [(sk0, si0), (sk1, si1)]
            for p in range(4):
                shiftv = jnp.full((16,), 8 * p, jnp.int32)
                if p > 0:
                    rk, ri = bufs[(p + 1) % 2]
                    pltpu.sync_copy(rk.at[pl.ds(base, _SORT_EPT)], key_v)
                    pltpu.sync_copy(ri.at[pl.ds(base, _SORT_EPT)], idx_v)
                wk, wi = bufs[p % 2]

                def zeroh(j, c):
                    hist_v[pl.ds(j * 16, 16)] = jnp.zeros((16,), jnp.int32)
                    return c
                lax.fori_loop(0, 16 * _NB // 16, zeroh, 0)

                def histb(v, c):
                    kv = key_v[pl.ds(v * 16, 16)]
                    d = jnp.bitwise_and(lax.shift_right_logical(kv, shiftv), 255)
                    plsc.addupdate_scatter(hist_v, [lane * _NB + d], ones)
                    return c
                lax.fori_loop(0, _SORT_V, histb, 0)

                def lred(c, carry):
                    def inner(l, acc):
                        return acc + hist_v[pl.ds(l * _NB + c * 16, 16)]
                    th_v[pl.ds(c * 16, 16)] = lax.fori_loop(
                        0, 16, inner, jnp.zeros((16,), jnp.int32))
                    return carry
                lax.fori_loop(0, _NB // 16, lred, 0)

                pltpu.sync_copy(th_v, gh.at[pl.ds(t * _NB, _NB)])
                plsc.subcore_barrier()
                pltpu.sync_copy(gh, ght_v)

                def totpre(c, carry):
                    def inner(l, tp):
                        tot, pre = tp
                        h = ght_v[pl.ds(l * _NB + c * 16, 16)]
                        return (tot + h, pre + h * jnp.where(l < t, 1, 0))
                    tot, pre = lax.fori_loop(
                        0, _SORT_T, inner,
                        (jnp.zeros((16,), jnp.int32), jnp.zeros((16,), jnp.int32)))
                    th_v[pl.ds(c * 16, 16)] = tot
                    cur_v[pl.ds(c * 16, 16)] = pre
                    return carry
                lax.fori_loop(0, _NB // 16, totpre, 0)

                def scan(c, s):
                    seg = th_v[pl.ds(c * 16, 16)]
                    inc = plsc.cumsum(seg)
                    cur_v[pl.ds(c * 16, 16)] = (
                        cur_v[pl.ds(c * 16, 16)] + (inc - seg) + s)
                    return s + jnp.sum(seg)
                lax.fori_loop(0, _NB // 16, scan, jnp.int32(0))

                for j in range(_SORT_EPT // 128):
                    def pbody(u, c, j=j):
                        v = j * 8 + u
                        kv = key_v[pl.ds(v * 16, 16)]
                        d = jnp.bitwise_and(
                            lax.shift_right_logical(kv, shiftv), 255)
                        w = jnp.zeros((16,), jnp.int32)
                        tot = jnp.zeros((16,), jnp.int32)
                        for u16 in range(16):
                            du = jnp.sum(jnp.where(lane == u16, d, 0))
                            m = jnp.where(d == du, 1, 0)
                            w = w + jnp.where(lane > u16, m, 0)
                            tot = tot + m
                        dest = plsc.load_gather(cur_v, [d]) + w
                        plsc.addupdate_scatter(cur_v, [d], w + 1,
                                               mask=(w + 1 == tot))
                        dst_v[j, pl.ds(u * 16, 16)] = dest
                        return c
                    lax.fori_loop(0, 8, pbody, 0)

                for j in range(_SORT_EPT // 128):
                    if p < 3:
                        pltpu.sync_copy(key_v.at[pl.ds(j * 128, 128)],
                                        wk.at[dst_v.at[j]])
                    pltpu.sync_copy(idx_v.at[pl.ds(j * 128, 128)],
                                    wi.at[dst_v.at[j]])
                plsc.subcore_barrier()

            pltpu.sync_copy(si1.at[pl.ds(base, _SORT_EPT)], idx_v)

            def clampi(v, c):
                sl = pl.ds(v * 16, 16)
                idx_v[sl] = jnp.minimum(idx_v[sl], jnp.int32(NUM_REL - 1))
                return c
            lax.fori_loop(0, _SORT_V, clampi, 0)
            pltpu.sync_copy(idx_v, out_hbm.at[pl.ds(base, _SORT_EPT)])

            def miscb(v, c):
                sl = pl.ds(v * 16, 16)
                se = idx_v[sl]
                p0s_v[sl] = plsc.load_gather(p0f, [se])
                p1s_v[sl] = plsc.load_gather(p1f, [se])
                lbls_v[sl] = plsc.load_gather(lblf, [se])
                return c
            lax.fori_loop(0, _SORT_V, miscb, 0)

            live = NUM_REL - (_SORT_T - 1) * _SORT_EPT

            @pl.when(t < _SORT_T - 1)
            def _mw():
                pltpu.sync_copy(p0s_v, p0s_hbm.at[pl.ds(base, _SORT_EPT)])
                pltpu.sync_copy(p1s_v, p1s_hbm.at[pl.ds(base, _SORT_EPT)])
                pltpu.sync_copy(lbls_v, lbls_hbm.at[pl.ds(base, _SORT_EPT)])

            @pl.when(t == _SORT_T - 1)
            def _mwt():
                pltpu.sync_copy(p0s_v.at[pl.ds(0, live)],
                                p0s_hbm.at[pl.ds(base, live)])
                pltpu.sync_copy(p1s_v.at[pl.ds(0, live)],
                                p1s_hbm.at[pl.ds(base, live)])
                pltpu.sync_copy(lbls_v.at[pl.ds(0, live)],
                                lbls_hbm.at[pl.ds(base, live)])

    return k(rs_pad, p0_pad, p1_pad, lbl_pad, osc_flat)


_NW = 32
_BPW = NUM_REL_P // _NW
_CHUNK = 128
_NCH = _BPW // _CHUNK


def _permute_rows(table, idx_1d):
    mesh = plsc.VectorSubcoreMesh(core_axis_name="c", subcore_axis_name="s")
    tail = NUM_REL - (_NW - 1) * _BPW

    @functools.partial(
        pl.kernel, mesh=mesh,
        out_type=jax.ShapeDtypeStruct((NUM_REL, ROW_W), jnp.float32),
        scratch_types=[
            pltpu.VMEM((_BPW,), jnp.int32),
            pltpu.VMEM((_BPW, ROW_W), jnp.float32),
            pltpu.SemaphoreType.DMA,
        ],
    )
    def k(table_hbm, idx_hbm, prob_hbm, idx_v, rows_v, sem):
        wid = lax.axis_index("s") * 2 + lax.axis_index("c")
        base = wid * _BPW
        pltpu.sync_copy(idx_hbm.at[pl.ds(base, _BPW)], idx_v)
        copies = [
            pltpu.async_copy(table_hbm.at[idx_v.at[pl.ds(j * _CHUNK, _CHUNK)]],
                             rows_v.at[pl.ds(j * _CHUNK, _CHUNK)], sem)
            for j in range(_NCH)
        ]
        for c in copies:
            c.wait()

        @pl.when(wid < _NW - 1)
        def _full():
            pltpu.sync_copy(rows_v, prob_hbm.at[pl.ds(base, _BPW)])

        @pl.when(wid == _NW - 1)
        def _tail():
            pltpu.sync_copy(rows_v.at[pl.ds(0, tail)],
                            prob_hbm.at[pl.ds(base, tail)])

    return k(table, idx_1d)


def kernel(rel_logit, obj_logit, rel_pair_idx):
    obj_logit_p = jnp.pad(obj_logit, ((0, NUM_OBJ_P - NUM_OBJ), (0, 0)))
    comb, rs, rcls, osc, opred = _prep(rel_logit, obj_logit_p, rel_pair_idx)
    obj_scores = osc.reshape(-1)[:NUM_OBJ]
    obj_pred = opred.reshape(-1)[:NUM_OBJ]

    pad = NUM_REL_P - NUM_REL
    rs_pad = jnp.concatenate([rs, jnp.full((pad,), -1.0, jnp.float32)])
    p0_pad = jnp.concatenate([rel_pair_idx[:, 0], jnp.zeros((pad,), jnp.int32)])
    p1_pad = jnp.concatenate([rel_pair_idx[:, 1], jnp.zeros((pad,), jnp.int32)])
    lbl_pad = jnp.concatenate([rcls, jnp.zeros((pad,), jnp.int32)])
    sorting_idx, p0s, p1s, rel_labels = _sc_argsort(
        rs_pad, p0_pad, p1_pad, lbl_pad, osc.reshape(-1))
    probs_full = _permute_rows(comb, sorting_idx)

    rel_class_prob_sorted = probs_full[:, :NUM_REL_CLS]
    rel_pair_idx_sorted = jnp.stack([p0s, p1s], axis=1)
    return (obj_pred, obj_scores, rel_pair_idx_sorted, rel_class_prob_sorted, rel_labels)
